# baseline (speedup 1.0000x reference)
import functools

import jax
import jax.numpy as jnp
from jax import lax
from jax.experimental import pallas as pl
from jax.experimental.pallas import tpu as pltpu
from jax.experimental.pallas import tpu_sc as plsc

_G = 512
_D = 32
_C = 4096
_NC = 2
_NS = 16
_L = 16
_CELLS_PER_SUB = _C // _NS
_GENES_PER_W = _G // (_NC * _NS)
_NBUF = 4


def _sc_body(sc_hbm, w_hbm, out_hbm, wstage, wt_chunk, row_buf, wt_spmem,
             slab_sems, row_sem):
    cid = lax.axis_index("c")
    sid = lax.axis_index("s")
    wid = sid * _NC + cid

    cell0 = sid * _CELLS_PER_SUB
    pltpu.sync_copy(w_hbm.at[pl.ds(cell0, _CELLS_PER_SUB), :], wstage)
    lane = lax.iota(jnp.int32, _L)
    for d in range(_D):
        d_idx = jnp.full((_L,), d, jnp.int32)
        for cgrp in range(_CELLS_PER_SUB // _L):
            c_idx = lane + (cgrp * _L)
            v = plsc.load_gather(wstage, [c_idx, d_idx])
            wt_chunk[d, pl.ds(cgrp * _L, _L)] = v
    pltpu.sync_copy(wt_chunk, wt_spmem.at[:, pl.ds(cell0, _CELLS_PER_SUB)])
    plsc.subcore_barrier()

    g0 = wid * _GENES_PER_W
    for k in range(_GENES_PER_W):
        g = g0 + k
        pltpu.async_copy(
            wt_spmem, out_hbm.at[g, pl.ds(0, _D), :], slab_sems.at[k % _NBUF]
        )
        if k >= _NBUF - 1:
            j = k - (_NBUF - 1)
            pltpu.make_async_copy(
                wt_spmem, out_hbm.at[g0 + j, pl.ds(0, _D), :],
                slab_sems.at[j % _NBUF],
            ).wait()
        pltpu.sync_copy(sc_hbm.at[pl.ds(g, 1), :], row_buf)
        pltpu.async_copy(row_buf, out_hbm.at[g, pl.ds(_D, 1), :], row_sem).wait()
    for j in range(_GENES_PER_W - (_NBUF - 1), _GENES_PER_W):
        pltpu.make_async_copy(
            wt_spmem, out_hbm.at[g0 + j, pl.ds(0, _D), :],
            slab_sems.at[j % _NBUF],
        ).wait()


def kernel(scRNA_count, embedding_weight):
    g, c = scRNA_count.shape
    c2, d = embedding_weight.shape
    assert (g, c, c2, d) == (_G, _C, _C, _D)

    mesh = plsc.VectorSubcoreMesh(core_axis_name="c", subcore_axis_name="s")
    f = functools.partial(
        pl.kernel,
        mesh=mesh,
        out_type=jax.ShapeDtypeStruct((_G, _D + 1, _C), jnp.float32),
        compiler_params=pltpu.CompilerParams(needs_layout_passes=False),
        scratch_types=[
            pltpu.VMEM((_CELLS_PER_SUB, _D), jnp.float32),
            pltpu.VMEM((_D, _CELLS_PER_SUB), jnp.float32),
            pltpu.VMEM((1, _C), jnp.float32),
            pltpu.VMEM_SHARED((_D, _C), jnp.float32),
            pltpu.SemaphoreType.DMA((_NBUF,)),
            pltpu.SemaphoreType.DMA,
        ],
    )(_sc_body)
    return f(scRNA_count, embedding_weight)
